# baseline (speedup 1.0000x reference)
import jax
import jax.numpy as jnp
from jax import lax
from jax.experimental import pallas as pl
from jax.experimental.pallas import tpu as pltpu
from jax.experimental.pallas import tpu_sc as plsc

NN = 10000
NE = 320000
DD = 128
HH = 64
NC = 2
NS = 16
SUB = 128
RP = 64
ROWS = NE // SUB
RPT = ROWS // NS
REM = ROWS - RPT * NS
APT = 624


def _prep_body(x_ref, awt, ab, bwt, bb, dwt, db, ewt, eb,
               ax_ref, dtab_ref, ebtab_ref):
    x = x_ref[...]
    ax_ref[...] = jnp.dot(x, awt[...], preferred_element_type=jnp.float32) + ab[...]
    dx = jnp.dot(x, dwt[...], preferred_element_type=jnp.float32) + db[...]
    ex = jnp.dot(x, ewt[...], preferred_element_type=jnp.float32) + eb[...]
    bx = jnp.dot(x, bwt[...], preferred_element_type=jnp.float32) + bb[...]
    dtab_ref[0] = dx
    dtab_ref[1] = jnp.concatenate([dx[:, HH:], dx[:, :HH]], axis=1)
    ebtab_ref[0] = jnp.concatenate([ex[:, :HH], bx[:, :HH]], axis=1)
    ebtab_ref[1] = jnp.concatenate([ex[:, HH:], bx[:, HH:]], axis=1)


def _prep(x, awt, ab, bwt, bb, dwt, db, ewt, eb):
    rb = 2000
    grid = NN // rb
    wspec = pl.BlockSpec((DD, DD), lambda i: (0, 0))
    bspec = pl.BlockSpec((1, DD), lambda i: (0, 0))
    return pl.pallas_call(
        _prep_body,
        grid=(grid,),
        in_specs=[pl.BlockSpec((rb, DD), lambda i: (i, 0)),
                  wspec, bspec, wspec, bspec, wspec, bspec, wspec, bspec],
        out_specs=[pl.BlockSpec((rb, DD), lambda i: (i, 0)),
                   pl.BlockSpec((NC, rb, DD), lambda i: (0, i, 0)),
                   pl.BlockSpec((NC, rb, DD), lambda i: (0, i, 0))],
        out_shape=[jax.ShapeDtypeStruct((NN, DD), jnp.float32),
                   jax.ShapeDtypeStruct((NC, NN, DD), jnp.float32),
                   jax.ShapeDtypeStruct((NC, NN, DD), jnp.float32)],
    )(x, awt, ab, bwt, bb, dwt, db, ewt, eb)


def _ce_body(ea_ref, cwt, cb, ce_ref):
    cea = jnp.dot(ea_ref[0], cwt[...], preferred_element_type=jnp.float32) + cb[...]
    ceb = jnp.dot(ea_ref[1], cwt[...], preferred_element_type=jnp.float32) + cb[...]
    ce_ref[0] = jnp.concatenate([cea[:, :HH], ceb[:, :HH]], axis=1)
    ce_ref[1] = jnp.concatenate([cea[:, HH:], ceb[:, HH:]], axis=1)


def _ce(ea2, cwt, cb):
    be = 2000
    grid = (NE // 2) // be
    return pl.pallas_call(
        _ce_body,
        grid=(grid,),
        in_specs=[pl.BlockSpec((2, be, DD), lambda i: (0, i, 0)),
                  pl.BlockSpec((DD, DD), lambda i: (0, 0)),
                  pl.BlockSpec((1, DD), lambda i: (0, 0))],
        out_specs=[pl.BlockSpec((NC, be, DD), lambda i: (0, i, 0))],
        out_shape=[jax.ShapeDtypeStruct((NC, NE // 2, DD), jnp.float32)],
    )(ea2, cwt, cb)[0]


def _sc_edge_body(dtab, ebtab, ce, sd3d,
                  eij, acc, stats,
                  sdb, gsrc, gdst, ebrows, cebuf,
                  msgbuf, statv, shacc, sem):
    c = lax.axis_index("c")
    s = lax.axis_index("s")
    cn = c * NN
    f0 = jnp.zeros((16,), jnp.float32)

    @pl.loop(0, SUB)
    def _zero_msg(i):
        for j in range(8):
            msgbuf[i, pl.ds(16 * j, 16)] = f0

    @pl.loop(0, 5)
    def _zero_q(q):
        pltpu.sync_copy(msgbuf, shacc.at[pl.ds(s * APT + q * SUB, SUB)])
    plsc.subcore_barrier()

    extra = jnp.where(s < REM, 1, 0)
    lo = s * RPT + jnp.minimum(s, REM)
    hi = lo + RPT + extra

    @pl.loop(lo, hi, init_carry=(f0,) * 8)
    def _chunk(r, stat_c):
        pltpu.sync_copy(sd3d.at[r], sdb)
        for j in range(8):
            sl = pl.ds(16 * j, 16)
            gsrc[sl] = sdb[0, sl] + cn
            gdst[sl] = sdb[1, sl] + cn
        cp1 = pltpu.async_copy(dtab.at[gdst], msgbuf, sem)
        cp2 = pltpu.async_copy(ebtab.at[gsrc], ebrows, sem)
        cp1.wait()
        cp2.wait()
        pltpu.sync_copy(ce.at[c, pl.ds(r * RP, RP)], cebuf)

        @plsc.parallel_loop(0, RP, carry=stat_c)
        def _row(p, cr):
            acc = list(cr)
            for half in range(2):
                i = p + HH * half
                for j in range(4):
                    sl = pl.ds(16 * j, 16)
                    sl2 = pl.ds(HH + 16 * j, 16)
                    cl = pl.ds(HH * half + 16 * j, 16)
                    e = msgbuf[i, sl] + ebrows[i, sl] + cebuf[p, cl]
                    cebuf[p, cl] = e
                    sg = 1.0 / (1.0 + jnp.exp(-e))
                    msgbuf[i, sl] = sg * ebrows[i, sl2]
                    msgbuf[i, sl2] = sg
                    acc[j] = acc[j] + e
                    acc[4 + j] = acc[4 + j] + e * e
            return tuple(acc)

        pltpu.sync_copy(cebuf, eij.at[c, pl.ds(r * RP, RP)])
        pltpu.sync_copy(msgbuf, shacc.at[sdb.at[1]], add=True)
        return _row

    for j in range(4):
        statv[0, pl.ds(16 * j, 16)] = _chunk[j]
        statv[0, pl.ds(HH + 16 * j, 16)] = _chunk[4 + j]
    pltpu.sync_copy(statv, stats.at[c, s])
    plsc.subcore_barrier()
    @pl.loop(0, 5)
    def _dump_q(q):
        pltpu.sync_copy(shacc.at[pl.ds(s * APT + q * SUB, SUB)], msgbuf)
        pltpu.sync_copy(msgbuf, acc.at[c, pl.ds(s * APT + q * SUB, SUB)])


def _sc_edge(dtab2, ebtab2, ce, sd3d):
    mesh = plsc.VectorSubcoreMesh(core_axis_name="c", subcore_axis_name="s",
                                  num_cores=NC, num_subcores=NS)
    f = pl.kernel(
        _sc_edge_body,
        compiler_params=pltpu.CompilerParams(use_tc_tiling_on_sc=False),
        out_type=[jax.ShapeDtypeStruct((NC, NE // 2, DD), jnp.float32),
                  jax.ShapeDtypeStruct((NC, NN, DD), jnp.float32),
                  jax.ShapeDtypeStruct((NC, NS, 8, DD), jnp.float32)],
        mesh=mesh,
        scratch_types=[
            pltpu.VMEM((2, SUB), jnp.int32),
            pltpu.VMEM((SUB,), jnp.int32),
            pltpu.VMEM((SUB,), jnp.int32),
            pltpu.VMEM((SUB, DD), jnp.float32),
            pltpu.VMEM((RP, DD), jnp.float32),
            pltpu.VMEM((SUB, DD), jnp.float32),
            pltpu.VMEM((8, DD), jnp.float32),
            pltpu.VMEM_SHARED((NN, DD), jnp.float32),
            pltpu.SemaphoreType.DMA,
        ],
    )
    return f(dtab2, ebtab2, ce, sd3d)


def _eout_body(eij_ref, ea_ref, st_ref, g_ref, b_ref, out_ref):
    st = st_ref[...]
    row0 = st[:, 0, :]
    c0 = jnp.sum(row0[:NS], axis=0)
    c1 = jnp.sum(row0[NS:], axis=0)
    mean = jnp.concatenate([c0[:HH], c1[:HH]]) * (1.0 / NE)
    msq = jnp.concatenate([c0[HH:], c1[HH:]]) * (1.0 / NE)
    var = msq - mean * mean
    scale = lax.rsqrt(var + 1e-5)
    ea = jnp.concatenate([eij_ref[0][:, :HH], eij_ref[1][:, :HH]], axis=1)
    eb = jnp.concatenate([eij_ref[0][:, HH:], eij_ref[1][:, HH:]], axis=1)

    def fin(e, attr):
        bn = (e - mean[None, :]) * scale[None, :] * g_ref[...] + b_ref[...]
        return attr + jnp.maximum(bn, 0.0)

    out_ref[0] = fin(ea, ea_ref[0])
    out_ref[1] = fin(eb, ea_ref[1])


def _eout(eij, ea2, stats4, g, b):
    be = 2000
    grid = (NE // 2) // be
    return pl.pallas_call(
        _eout_body,
        grid=(grid,),
        in_specs=[pl.BlockSpec((NC, be, DD), lambda i: (0, i, 0)),
                  pl.BlockSpec((2, be, DD), lambda i: (0, i, 0)),
                  pl.BlockSpec((NC * NS, 8, DD), lambda i: (0, 0, 0)),
                  pl.BlockSpec((1, DD), lambda i: (0, 0)),
                  pl.BlockSpec((1, DD), lambda i: (0, 0))],
        out_specs=pl.BlockSpec((2, be, DD), lambda i: (0, i, 0)),
        out_shape=jax.ShapeDtypeStruct((2, NE // 2, DD), jnp.float32),
    )(eij, ea2, stats4, g, b)


def _bn_tc(v, g, b):
    m = jnp.mean(v, axis=0, keepdims=True)
    var = jnp.mean(v * v, axis=0, keepdims=True) - m * m
    return (v - m) * lax.rsqrt(var + 1e-5) * g + b


def _x_body(x_ref, ax_ref, acc_ref, bxg, bxb, n1g, n1b, n2g, n2b,
            f1wt, f1b, f2wt, f2b, out_ref):
    num = jnp.concatenate([acc_ref[0][:, :HH], acc_ref[1][:, :HH]], axis=1)
    den = jnp.concatenate([acc_ref[0][:, HH:], acc_ref[1][:, HH:]], axis=1)
    h = ax_ref[...] + num / (den + 1e-6)
    h = jnp.maximum(_bn_tc(h, bxg[...], bxb[...]), 0.0)
    h = x_ref[...] + h
    h = _bn_tc(h, n1g[...], n1b[...])
    ff = jnp.maximum(
        jnp.dot(h, f1wt[...], preferred_element_type=jnp.float32) + f1b[...], 0.0)
    h2 = h + jnp.dot(ff, f2wt[...], preferred_element_type=jnp.float32) + f2b[...]
    out_ref[...] = _bn_tc(h2, n2g[...], n2b[...])


def _xpipe(x, ax, acc, bxg, bxb, n1g, n1b, n2g, n2b, f1wt, f1b, f2wt, f2b):
    return pl.pallas_call(
        _x_body,
        out_shape=jax.ShapeDtypeStruct((NN, DD), jnp.float32),
    )(x, ax, acc, bxg, bxb, n1g, n1b, n2g, n2b, f1wt, f1b, f2wt, f2b)


def kernel(x, edge_attr, edge_index, A_w, A_b, B_w, B_b, C_w, C_b, D_w, D_b,
           E_w, E_b, ff1_w, ff1_b, ff2_w, ff2_b, bnx_g, bnx_b, bne_g, bne_b,
           n1_g, n1_b, n2_g, n2_b):
    r1 = lambda v: v.reshape(1, -1)
    sd3d = (edge_index.reshape(2, 2, ROWS, RP)
            .transpose(2, 0, 1, 3).reshape(ROWS, 2, SUB))

    ea2 = edge_attr.reshape(2, NE // 2, DD)
    ax, dtab, ebtab = _prep(x, A_w.T, r1(A_b), B_w.T, r1(B_b),
                            D_w.T, r1(D_b), E_w.T, r1(E_b))
    ce = _ce(ea2, C_w.T, r1(C_b))

    eij, acc, stats = _sc_edge(dtab.reshape(NC * NN, DD),
                               ebtab.reshape(NC * NN, DD),
                               ce, sd3d)

    e_out = _eout(eij, ea2, stats.reshape(NC * NS, 8, DD),
                  r1(bne_g), r1(bne_b)).reshape(NE, DD)
    x_out = _xpipe(x, ax, acc, r1(bnx_g), r1(bnx_b), r1(n1_g), r1(n1_b),
                   r1(n2_g), r1(n2_b), ff1_w.T, r1(ff1_b), ff2_w.T, r1(ff2_b))
    return (x_out, e_out)


# speedup vs baseline: 1.2425x; 1.2425x over previous
import jax
import jax.numpy as jnp
from jax import lax
from jax.experimental import pallas as pl
from jax.experimental.pallas import tpu as pltpu
from jax.experimental.pallas import tpu_sc as plsc

NN = 10000
NE = 320000
DD = 128
HH = 64
NC = 2
NS = 16
SUB = 64
RP = 32
ROWS = NE // SUB
PAIRS = ROWS // 2
PPT = PAIRS // NS
PREM = PAIRS - PPT * NS
APT = 624


def _prep_body(x_ref, awt, ab, bwt, bb, dwt, db, ewt, eb,
               ax_ref, dtab_ref, ebtab_ref):
    x = x_ref[...]
    ax_ref[...] = jnp.dot(x, awt[...], preferred_element_type=jnp.float32) + ab[...]
    dx = jnp.dot(x, dwt[...], preferred_element_type=jnp.float32) + db[...]
    ex = jnp.dot(x, ewt[...], preferred_element_type=jnp.float32) + eb[...]
    bx = jnp.dot(x, bwt[...], preferred_element_type=jnp.float32) + bb[...]
    dtab_ref[0] = dx
    dtab_ref[1] = jnp.concatenate([dx[:, HH:], dx[:, :HH]], axis=1)
    ebtab_ref[0] = jnp.concatenate([ex[:, :HH], bx[:, :HH]], axis=1)
    ebtab_ref[1] = jnp.concatenate([ex[:, HH:], bx[:, HH:]], axis=1)


def _prep(x, awt, ab, bwt, bb, dwt, db, ewt, eb):
    rb = 2000
    grid = NN // rb
    wspec = pl.BlockSpec((DD, DD), lambda i: (0, 0))
    bspec = pl.BlockSpec((1, DD), lambda i: (0, 0))
    return pl.pallas_call(
        _prep_body,
        grid=(grid,),
        in_specs=[pl.BlockSpec((rb, DD), lambda i: (i, 0)),
                  wspec, bspec, wspec, bspec, wspec, bspec, wspec, bspec],
        out_specs=[pl.BlockSpec((rb, DD), lambda i: (i, 0)),
                   pl.BlockSpec((NC, rb, DD), lambda i: (0, i, 0)),
                   pl.BlockSpec((NC, rb, DD), lambda i: (0, i, 0))],
        out_shape=[jax.ShapeDtypeStruct((NN, DD), jnp.float32),
                   jax.ShapeDtypeStruct((NC, NN, DD), jnp.float32),
                   jax.ShapeDtypeStruct((NC, NN, DD), jnp.float32)],
    )(x, awt, ab, bwt, bb, dwt, db, ewt, eb)


def _ce_body(ea_ref, cwt, cb, ce_ref):
    cea = jnp.dot(ea_ref[0], cwt[...], preferred_element_type=jnp.float32) + cb[...]
    ceb = jnp.dot(ea_ref[1], cwt[...], preferred_element_type=jnp.float32) + cb[...]
    ce_ref[0] = jnp.concatenate([cea[:, :HH], ceb[:, :HH]], axis=1)
    ce_ref[1] = jnp.concatenate([cea[:, HH:], ceb[:, HH:]], axis=1)


def _ce(ea2, cwt, cb):
    be = 2000
    grid = (NE // 2) // be
    return pl.pallas_call(
        _ce_body,
        grid=(grid,),
        in_specs=[pl.BlockSpec((2, be, DD), lambda i: (0, i, 0)),
                  pl.BlockSpec((DD, DD), lambda i: (0, 0)),
                  pl.BlockSpec((1, DD), lambda i: (0, 0))],
        out_specs=[pl.BlockSpec((NC, be, DD), lambda i: (0, i, 0))],
        out_shape=[jax.ShapeDtypeStruct((NC, NE // 2, DD), jnp.float32)],
    )(ea2, cwt, cb)[0]


def _sc_edge_body(dtab, ebtab, ce, sd3d,
                  eij, acc, stats,
                  sdb0, gsrc0, gdst0, ebrows0, cebuf0, msgbuf0,
                  sdb1, gsrc1, gdst1, ebrows1, cebuf1, msgbuf1,
                  statv, shacc,
                  semg0, semc0, seme0, sems0, semg1, semc1, seme1, sems1):
    c = lax.axis_index("c")
    s = lax.axis_index("s")
    cn = c * NN
    f0 = jnp.zeros((16,), jnp.float32)

    buf0 = (sdb0, gsrc0, gdst0, ebrows0, cebuf0, msgbuf0, semg0, semc0, seme0, sems0)
    buf1 = (sdb1, gsrc1, gdst1, ebrows1, cebuf1, msgbuf1, semg1, semc1, seme1, sems1)

    @pl.loop(0, SUB)
    def _zero_msg(i):
        for j in range(8):
            msgbuf0[i, pl.ds(16 * j, 16)] = f0

    @pl.loop(0, 10)
    def _zero_q(q):
        pltpu.sync_copy(msgbuf0, shacc.at[pl.ds(s * APT + q * SUB, SUB)])
    plsc.subcore_barrier()

    def fire_g(r, buf):
        sdb, gsrc, gdst, ebrows, cebuf, msgbuf, semg, semc = buf[:8]
        pltpu.sync_copy(sd3d.at[r], sdb)
        for j in range(4):
            sl = pl.ds(16 * j, 16)
            gsrc[sl] = sdb[0, sl] + cn
            gdst[sl] = sdb[1, sl] + cn
        cp1 = pltpu.async_copy(dtab.at[gdst], msgbuf, semg)
        cp2 = pltpu.async_copy(ebtab.at[gsrc], ebrows, semg)
        cp3 = pltpu.async_copy(ce.at[c, pl.ds(r * RP, RP)], cebuf, semc)
        return (cp1, cp2, cp3)

    def fire_w(r, buf):
        sdb, cebuf, msgbuf, seme, sems = buf[0], buf[4], buf[5], buf[8], buf[9]
        cp1 = pltpu.async_copy(cebuf, eij.at[c, pl.ds(r * RP, RP)], seme)
        cp2 = pltpu.async_copy(msgbuf, shacc.at[sdb.at[1]], sems, add=True)
        return (cp1, cp2)

    def compute(buf, stat_c):
        ebrows, cebuf, msgbuf = buf[3], buf[4], buf[5]

        @plsc.parallel_loop(0, RP, carry=stat_c)
        def _row(p, cr):
            acc_ = list(cr)
            for half in range(2):
                i = p + RP * half
                for j in range(4):
                    sl = pl.ds(16 * j, 16)
                    sl2 = pl.ds(HH + 16 * j, 16)
                    cl = pl.ds(HH * half + 16 * j, 16)
                    e = msgbuf[i, sl] + ebrows[i, sl] + cebuf[p, cl]
                    cebuf[p, cl] = e
                    sg = 1.0 / (1.0 + jnp.exp(-e))
                    msgbuf[i, sl] = sg * ebrows[i, sl2]
                    msgbuf[i, sl2] = sg
                    acc_[j] = acc_[j] + e
                    acc_[4 + j] = acc_[4 + j] + e * e
            return tuple(acc_)

        return _row

    plo = s * PPT + jnp.minimum(s, PREM)
    phi = plo + PPT + jnp.where(s < PREM, 1, 0)

    @pl.loop(plo, phi, init_carry=(f0,) * 8)
    def _pair(g, stat_c):
        r0 = 2 * g
        r1 = 2 * g + 1
        g0 = fire_g(r0, buf0)
        g1 = fire_g(r1, buf1)
        for cp in g0:
            cp.wait()
        st1 = compute(buf0, stat_c)
        w0 = fire_w(r0, buf0)
        for cp in g1:
            cp.wait()
        st2 = compute(buf1, st1)
        w1 = fire_w(r1, buf1)
        for cp in w0 + w1:
            cp.wait()
        return st2

    for j in range(4):
        statv[0, pl.ds(16 * j, 16)] = _pair[j]
        statv[0, pl.ds(HH + 16 * j, 16)] = _pair[4 + j]
    pltpu.sync_copy(statv, stats.at[c, s])
    plsc.subcore_barrier()
    @pl.loop(0, 10)
    def _dump_q(q):
        pltpu.sync_copy(shacc.at[pl.ds(s * APT + q * SUB, SUB)], msgbuf0)
        pltpu.sync_copy(msgbuf0, acc.at[c, pl.ds(s * APT + q * SUB, SUB)])

    @pl.when(s == NS - 1)
    def _acc_tail():
        pltpu.sync_copy(shacc.at[pl.ds(NS * APT, 16)],
                        msgbuf0.at[pl.ds(0, 16)])
        pltpu.sync_copy(msgbuf0.at[pl.ds(0, 16)],
                        acc.at[c, pl.ds(NS * APT, 16)])


def _sc_edge(dtab2, ebtab2, ce, sd3d):
    mesh = plsc.VectorSubcoreMesh(core_axis_name="c", subcore_axis_name="s",
                                  num_cores=NC, num_subcores=NS)
    vb = [
        pltpu.VMEM((2, SUB), jnp.int32),
        pltpu.VMEM((SUB,), jnp.int32),
        pltpu.VMEM((SUB,), jnp.int32),
        pltpu.VMEM((SUB, DD), jnp.float32),
        pltpu.VMEM((RP, DD), jnp.float32),
        pltpu.VMEM((SUB, DD), jnp.float32),
    ]
    f = pl.kernel(
        _sc_edge_body,
        compiler_params=pltpu.CompilerParams(use_tc_tiling_on_sc=False),
        out_type=[jax.ShapeDtypeStruct((NC, NE // 2, DD), jnp.float32),
                  jax.ShapeDtypeStruct((NC, NN, DD), jnp.float32),
                  jax.ShapeDtypeStruct((NC, NS, 8, DD), jnp.float32)],
        mesh=mesh,
        scratch_types=vb + vb + [
            pltpu.VMEM((8, DD), jnp.float32),
            pltpu.VMEM_SHARED((NN, DD), jnp.float32),
        ] + [pltpu.SemaphoreType.DMA] * 8,
    )
    return f(dtab2, ebtab2, ce, sd3d)


def _eout_body(eij_ref, ea_ref, st_ref, g_ref, b_ref, out_ref):
    st = st_ref[...]
    row0 = st[:, 0, :]
    c0 = jnp.sum(row0[:NS], axis=0)
    c1 = jnp.sum(row0[NS:], axis=0)
    mean = jnp.concatenate([c0[:HH], c1[:HH]]) * (1.0 / NE)
    msq = jnp.concatenate([c0[HH:], c1[HH:]]) * (1.0 / NE)
    var = msq - mean * mean
    scale = lax.rsqrt(var + 1e-5)
    ea = jnp.concatenate([eij_ref[0][:, :HH], eij_ref[1][:, :HH]], axis=1)
    eb = jnp.concatenate([eij_ref[0][:, HH:], eij_ref[1][:, HH:]], axis=1)

    def fin(e, attr):
        bn = (e - mean[None, :]) * scale[None, :] * g_ref[...] + b_ref[...]
        return attr + jnp.maximum(bn, 0.0)

    out_ref[0] = fin(ea, ea_ref[0])
    out_ref[1] = fin(eb, ea_ref[1])


def _eout(eij, ea2, stats4, g, b):
    be = 2000
    grid = (NE // 2) // be
    return pl.pallas_call(
        _eout_body,
        grid=(grid,),
        in_specs=[pl.BlockSpec((NC, be, DD), lambda i: (0, i, 0)),
                  pl.BlockSpec((2, be, DD), lambda i: (0, i, 0)),
                  pl.BlockSpec((NC * NS, 8, DD), lambda i: (0, 0, 0)),
                  pl.BlockSpec((1, DD), lambda i: (0, 0)),
                  pl.BlockSpec((1, DD), lambda i: (0, 0))],
        out_specs=pl.BlockSpec((2, be, DD), lambda i: (0, i, 0)),
        out_shape=jax.ShapeDtypeStruct((2, NE // 2, DD), jnp.float32),
    )(eij, ea2, stats4, g, b)


def _bn_tc(v, g, b):
    m = jnp.mean(v, axis=0, keepdims=True)
    var = jnp.mean(v * v, axis=0, keepdims=True) - m * m
    return (v - m) * lax.rsqrt(var + 1e-5) * g + b


def _x_body(x_ref, ax_ref, acc_ref, bxg, bxb, n1g, n1b, n2g, n2b,
            f1wt, f1b, f2wt, f2b, out_ref):
    num = jnp.concatenate([acc_ref[0][:, :HH], acc_ref[1][:, :HH]], axis=1)
    den = jnp.concatenate([acc_ref[0][:, HH:], acc_ref[1][:, HH:]], axis=1)
    h = ax_ref[...] + num / (den + 1e-6)
    h = jnp.maximum(_bn_tc(h, bxg[...], bxb[...]), 0.0)
    h = x_ref[...] + h
    h = _bn_tc(h, n1g[...], n1b[...])
    ff = jnp.maximum(
        jnp.dot(h, f1wt[...], preferred_element_type=jnp.float32) + f1b[...], 0.0)
    h2 = h + jnp.dot(ff, f2wt[...], preferred_element_type=jnp.float32) + f2b[...]
    out_ref[...] = _bn_tc(h2, n2g[...], n2b[...])


def _xpipe(x, ax, acc, bxg, bxb, n1g, n1b, n2g, n2b, f1wt, f1b, f2wt, f2b):
    return pl.pallas_call(
        _x_body,
        out_shape=jax.ShapeDtypeStruct((NN, DD), jnp.float32),
    )(x, ax, acc, bxg, bxb, n1g, n1b, n2g, n2b, f1wt, f1b, f2wt, f2b)


def kernel(x, edge_attr, edge_index, A_w, A_b, B_w, B_b, C_w, C_b, D_w, D_b,
           E_w, E_b, ff1_w, ff1_b, ff2_w, ff2_b, bnx_g, bnx_b, bne_g, bne_b,
           n1_g, n1_b, n2_g, n2_b):
    r1 = lambda v: v.reshape(1, -1)
    sd3d = (edge_index.reshape(2, 2, ROWS, RP)
            .transpose(2, 0, 1, 3).reshape(ROWS, 2, SUB))

    ea2 = edge_attr.reshape(2, NE // 2, DD)
    ax, dtab, ebtab = _prep(x, A_w.T, r1(A_b), B_w.T, r1(B_b),
                            D_w.T, r1(D_b), E_w.T, r1(E_b))
    ce = _ce(ea2, C_w.T, r1(C_b))

    eij, acc, stats = _sc_edge(dtab.reshape(NC * NN, DD),
                               ebtab.reshape(NC * NN, DD),
                               ce, sd3d)

    e_out = _eout(eij, ea2, stats.reshape(NC * NS, 8, DD),
                  r1(bne_g), r1(bne_b)).reshape(NE, DD)
    x_out = _xpipe(x, ax, acc, r1(bnx_g), r1(bnx_b), r1(n1_g), r1(n1_b),
                   r1(n2_g), r1(n2_b), ff1_w.T, r1(ff1_b), ff2_w.T, r1(ff2_b))
    return (x_out, e_out)


# speedup vs baseline: 1.2429x; 1.0003x over previous
import jax
import jax.numpy as jnp
from jax import lax
from jax.experimental import pallas as pl
from jax.experimental.pallas import tpu as pltpu
from jax.experimental.pallas import tpu_sc as plsc

NN = 10000
NE = 320000
DD = 128
HH = 64
NC = 2
NS = 16
SUB = 64
RP = 32
ROWS = NE // SUB
PAIRS = ROWS // 2
PPT = PAIRS // NS
PREM = PAIRS - PPT * NS
APT = 624


def _prep_body(x_ref, awt, ab, bwt, bb, dwt, db, ewt, eb,
               ax_ref, dtab_ref, ebtab_ref):
    x = x_ref[...]
    ax_ref[...] = jnp.dot(x, awt[...], preferred_element_type=jnp.float32) + ab[...]
    dx = jnp.dot(x, dwt[...], preferred_element_type=jnp.float32) + db[...]
    ex = jnp.dot(x, ewt[...], preferred_element_type=jnp.float32) + eb[...]
    bx = jnp.dot(x, bwt[...], preferred_element_type=jnp.float32) + bb[...]
    dtab_ref[0] = dx
    dtab_ref[1] = jnp.concatenate([dx[:, HH:], dx[:, :HH]], axis=1)
    ebtab_ref[0] = jnp.concatenate([ex[:, :HH], bx[:, :HH]], axis=1)
    ebtab_ref[1] = jnp.concatenate([ex[:, HH:], bx[:, HH:]], axis=1)


def _prep(x, awt, ab, bwt, bb, dwt, db, ewt, eb):
    rb = 2000
    grid = NN // rb
    wspec = pl.BlockSpec((DD, DD), lambda i: (0, 0))
    bspec = pl.BlockSpec((1, DD), lambda i: (0, 0))
    return pl.pallas_call(
        _prep_body,
        grid=(grid,),
        in_specs=[pl.BlockSpec((rb, DD), lambda i: (i, 0)),
                  wspec, bspec, wspec, bspec, wspec, bspec, wspec, bspec],
        out_specs=[pl.BlockSpec((rb, DD), lambda i: (i, 0)),
                   pl.BlockSpec((NC, rb, DD), lambda i: (0, i, 0)),
                   pl.BlockSpec((NC, rb, DD), lambda i: (0, i, 0))],
        out_shape=[jax.ShapeDtypeStruct((NN, DD), jnp.float32),
                   jax.ShapeDtypeStruct((NC, NN, DD), jnp.float32),
                   jax.ShapeDtypeStruct((NC, NN, DD), jnp.float32)],
    )(x, awt, ab, bwt, bb, dwt, db, ewt, eb)


def _ce_body(ea_ref, cwt, cb, ce_ref):
    cea = jnp.dot(ea_ref[0], cwt[...], preferred_element_type=jnp.float32) + cb[...]
    ceb = jnp.dot(ea_ref[1], cwt[...], preferred_element_type=jnp.float32) + cb[...]
    ce_ref[0] = jnp.concatenate([cea[:, :HH], ceb[:, :HH]], axis=1)
    ce_ref[1] = jnp.concatenate([cea[:, HH:], ceb[:, HH:]], axis=1)


def _ce(ea2, cwt, cb):
    be = 2000
    grid = (NE // 2) // be
    return pl.pallas_call(
        _ce_body,
        grid=(grid,),
        in_specs=[pl.BlockSpec((2, be, DD), lambda i: (0, i, 0)),
                  pl.BlockSpec((DD, DD), lambda i: (0, 0)),
                  pl.BlockSpec((1, DD), lambda i: (0, 0))],
        out_specs=[pl.BlockSpec((NC, be, DD), lambda i: (0, i, 0))],
        out_shape=[jax.ShapeDtypeStruct((NC, NE // 2, DD), jnp.float32)],
    )(ea2, cwt, cb)[0]


def _sc_edge_body(dtab, ebtab, ce, sd3d,
                  eij, acc, stats,
                  sdb0, gsrc0, gdst0, ebrows0, cebuf0, msgbuf0,
                  sdb1, gsrc1, gdst1, ebrows1, cebuf1, msgbuf1,
                  statv, shacc,
                  semg0, semc0, seme0, sems0, semg1, semc1, seme1, sems1):
    c = lax.axis_index("c")
    s = lax.axis_index("s")
    cn = c * NN
    f0 = jnp.zeros((16,), jnp.float32)

    buf0 = (sdb0, gsrc0, gdst0, ebrows0, cebuf0, msgbuf0, semg0, semc0, seme0, sems0)
    buf1 = (sdb1, gsrc1, gdst1, ebrows1, cebuf1, msgbuf1, semg1, semc1, seme1, sems1)

    @pl.loop(0, SUB)
    def _zero_msg(i):
        for j in range(8):
            msgbuf0[i, pl.ds(16 * j, 16)] = f0

    @pl.loop(0, 10)
    def _zero_q(q):
        pltpu.sync_copy(msgbuf0, shacc.at[pl.ds(s * APT + q * SUB, SUB)])
    plsc.subcore_barrier()

    def fire_g(r, buf):
        sdb, gsrc, gdst, ebrows, cebuf, msgbuf, semg, semc = buf[:8]
        pltpu.sync_copy(sd3d.at[r], sdb)
        for j in range(4):
            sl = pl.ds(16 * j, 16)
            gsrc[sl] = sdb[0, sl] + cn
            gdst[sl] = sdb[1, sl] + cn
        cp1 = pltpu.async_copy(dtab.at[gdst], msgbuf, semg)
        cp2 = pltpu.async_copy(ebtab.at[gsrc], ebrows, semg)
        cp3 = pltpu.async_copy(ce.at[c, pl.ds(r * RP, RP)], cebuf, semc)
        return (cp1, cp2, cp3)

    def fire_w(r, buf):
        sdb, cebuf, msgbuf, seme, sems = buf[0], buf[4], buf[5], buf[8], buf[9]
        cp1 = pltpu.async_copy(cebuf, eij.at[c, pl.ds(r * RP, RP)], seme)
        cp2 = pltpu.async_copy(msgbuf, shacc.at[sdb.at[1]], sems, add=True)
        return (cp1, cp2)

    def compute(buf, stat_c):
        ebrows, cebuf, msgbuf = buf[3], buf[4], buf[5]

        @plsc.parallel_loop(0, RP, unroll=2, carry=stat_c)
        def _row(p, cr):
            acc_ = list(cr)
            for half in range(2):
                i = p + RP * half
                for j in range(4):
                    sl = pl.ds(16 * j, 16)
                    sl2 = pl.ds(HH + 16 * j, 16)
                    cl = pl.ds(HH * half + 16 * j, 16)
                    e = msgbuf[i, sl] + ebrows[i, sl] + cebuf[p, cl]
                    cebuf[p, cl] = e
                    sg = 1.0 / (1.0 + jnp.exp(-e))
                    msgbuf[i, sl] = sg * ebrows[i, sl2]
                    msgbuf[i, sl2] = sg
                    acc_[j] = acc_[j] + e
                    acc_[4 + j] = acc_[4 + j] + e * e
            return tuple(acc_)

        return _row

    plo = s * PPT + jnp.minimum(s, PREM)
    phi = plo + PPT + jnp.where(s < PREM, 1, 0)

    @pl.loop(plo, phi, init_carry=(f0,) * 8)
    def _pair(g, stat_c):
        r0 = 2 * g
        r1 = 2 * g + 1
        g0 = fire_g(r0, buf0)
        g1 = fire_g(r1, buf1)
        for cp in g0:
            cp.wait()
        st1 = compute(buf0, stat_c)
        w0 = fire_w(r0, buf0)
        for cp in g1:
            cp.wait()
        st2 = compute(buf1, st1)
        w1 = fire_w(r1, buf1)
        for cp in w0 + w1:
            cp.wait()
        return st2

    for j in range(4):
        statv[0, pl.ds(16 * j, 16)] = _pair[j]
        statv[0, pl.ds(HH + 16 * j, 16)] = _pair[4 + j]
    pltpu.sync_copy(statv, stats.at[c, s])
    plsc.subcore_barrier()
    @pl.loop(0, 10)
    def _dump_q(q):
        pltpu.sync_copy(shacc.at[pl.ds(s * APT + q * SUB, SUB)], msgbuf0)
        pltpu.sync_copy(msgbuf0, acc.at[c, pl.ds(s * APT + q * SUB, SUB)])

    @pl.when(s == NS - 1)
    def _acc_tail():
        pltpu.sync_copy(shacc.at[pl.ds(NS * APT, 16)],
                        msgbuf0.at[pl.ds(0, 16)])
        pltpu.sync_copy(msgbuf0.at[pl.ds(0, 16)],
                        acc.at[c, pl.ds(NS * APT, 16)])


def _sc_edge(dtab2, ebtab2, ce, sd3d):
    mesh = plsc.VectorSubcoreMesh(core_axis_name="c", subcore_axis_name="s",
                                  num_cores=NC, num_subcores=NS)
    vb = [
        pltpu.VMEM((2, SUB), jnp.int32),
        pltpu.VMEM((SUB,), jnp.int32),
        pltpu.VMEM((SUB,), jnp.int32),
        pltpu.VMEM((SUB, DD), jnp.float32),
        pltpu.VMEM((RP, DD), jnp.float32),
        pltpu.VMEM((SUB, DD), jnp.float32),
    ]
    f = pl.kernel(
        _sc_edge_body,
        compiler_params=pltpu.CompilerParams(use_tc_tiling_on_sc=False),
        out_type=[jax.ShapeDtypeStruct((NC, NE // 2, DD), jnp.float32),
                  jax.ShapeDtypeStruct((NC, NN, DD), jnp.float32),
                  jax.ShapeDtypeStruct((NC, NS, 8, DD), jnp.float32)],
        mesh=mesh,
        scratch_types=vb + vb + [
            pltpu.VMEM((8, DD), jnp.float32),
            pltpu.VMEM_SHARED((NN, DD), jnp.float32),
        ] + [pltpu.SemaphoreType.DMA] * 8,
    )
    return f(dtab2, ebtab2, ce, sd3d)


def _eout_body(eij_ref, ea_ref, st_ref, g_ref, b_ref, out_ref):
    st = st_ref[...]
    row0 = st[:, 0, :]
    c0 = jnp.sum(row0[:NS], axis=0)
    c1 = jnp.sum(row0[NS:], axis=0)
    mean = jnp.concatenate([c0[:HH], c1[:HH]]) * (1.0 / NE)
    msq = jnp.concatenate([c0[HH:], c1[HH:]]) * (1.0 / NE)
    var = msq - mean * mean
    scale = lax.rsqrt(var + 1e-5)
    ea = jnp.concatenate([eij_ref[0][:, :HH], eij_ref[1][:, :HH]], axis=1)
    eb = jnp.concatenate([eij_ref[0][:, HH:], eij_ref[1][:, HH:]], axis=1)

    def fin(e, attr):
        bn = (e - mean[None, :]) * scale[None, :] * g_ref[...] + b_ref[...]
        return attr + jnp.maximum(bn, 0.0)

    out_ref[0] = fin(ea, ea_ref[0])
    out_ref[1] = fin(eb, ea_ref[1])


def _eout(eij, ea2, stats4, g, b):
    be = 2000
    grid = (NE // 2) // be
    return pl.pallas_call(
        _eout_body,
        grid=(grid,),
        in_specs=[pl.BlockSpec((NC, be, DD), lambda i: (0, i, 0)),
                  pl.BlockSpec((2, be, DD), lambda i: (0, i, 0)),
                  pl.BlockSpec((NC * NS, 8, DD), lambda i: (0, 0, 0)),
                  pl.BlockSpec((1, DD), lambda i: (0, 0)),
                  pl.BlockSpec((1, DD), lambda i: (0, 0))],
        out_specs=pl.BlockSpec((2, be, DD), lambda i: (0, i, 0)),
        out_shape=jax.ShapeDtypeStruct((2, NE // 2, DD), jnp.float32),
    )(eij, ea2, stats4, g, b)


def _bn_tc(v, g, b):
    m = jnp.mean(v, axis=0, keepdims=True)
    var = jnp.mean(v * v, axis=0, keepdims=True) - m * m
    return (v - m) * lax.rsqrt(var + 1e-5) * g + b


def _x_body(x_ref, ax_ref, acc_ref, bxg, bxb, n1g, n1b, n2g, n2b,
            f1wt, f1b, f2wt, f2b, out_ref):
    num = jnp.concatenate([acc_ref[0][:, :HH], acc_ref[1][:, :HH]], axis=1)
    den = jnp.concatenate([acc_ref[0][:, HH:], acc_ref[1][:, HH:]], axis=1)
    h = ax_ref[...] + num / (den + 1e-6)
    h = jnp.maximum(_bn_tc(h, bxg[...], bxb[...]), 0.0)
    h = x_ref[...] + h
    h = _bn_tc(h, n1g[...], n1b[...])
    ff = jnp.maximum(
        jnp.dot(h, f1wt[...], preferred_element_type=jnp.float32) + f1b[...], 0.0)
    h2 = h + jnp.dot(ff, f2wt[...], preferred_element_type=jnp.float32) + f2b[...]
    out_ref[...] = _bn_tc(h2, n2g[...], n2b[...])


def _xpipe(x, ax, acc, bxg, bxb, n1g, n1b, n2g, n2b, f1wt, f1b, f2wt, f2b):
    return pl.pallas_call(
        _x_body,
        out_shape=jax.ShapeDtypeStruct((NN, DD), jnp.float32),
    )(x, ax, acc, bxg, bxb, n1g, n1b, n2g, n2b, f1wt, f1b, f2wt, f2b)


def kernel(x, edge_attr, edge_index, A_w, A_b, B_w, B_b, C_w, C_b, D_w, D_b,
           E_w, E_b, ff1_w, ff1_b, ff2_w, ff2_b, bnx_g, bnx_b, bne_g, bne_b,
           n1_g, n1_b, n2_g, n2_b):
    r1 = lambda v: v.reshape(1, -1)
    sd3d = (edge_index.reshape(2, 2, ROWS, RP)
            .transpose(2, 0, 1, 3).reshape(ROWS, 2, SUB))

    ea2 = edge_attr.reshape(2, NE // 2, DD)
    ax, dtab, ebtab = _prep(x, A_w.T, r1(A_b), B_w.T, r1(B_b),
                            D_w.T, r1(D_b), E_w.T, r1(E_b))
    ce = _ce(ea2, C_w.T, r1(C_b))

    eij, acc, stats = _sc_edge(dtab.reshape(NC * NN, DD),
                               ebtab.reshape(NC * NN, DD),
                               ce, sd3d)

    e_out = _eout(eij, ea2, stats.reshape(NC * NS, 8, DD),
                  r1(bne_g), r1(bne_b)).reshape(NE, DD)
    x_out = _xpipe(x, ax, acc, r1(bnx_g), r1(bnx_b), r1(n1_g), r1(n1_b),
                   r1(n2_g), r1(n2_b), ff1_w.T, r1(ff1_b), ff2_w.T, r1(ff2_b))
    return (x_out, e_out)
